# baseline (speedup 1.0000x reference)
import jax
import jax.numpy as jnp
from jax import lax
from jax.experimental import pallas as pl
from jax.experimental.pallas import tpu as pltpu
from jax.experimental.pallas import tpu_sc as plsc

_B = 1024
_L = 200
_D = 64

_NC = 2
_NS = 16
_NW = _NC * _NS

_BPW = _B // _NW
_SPLITS = ((0, 104), (104, 96))


def _body(ids_hbm, table_hbm, pos_hbm, out_hbm,
          idx_nat, idx_bm, idx_pair, rows0, rows1, st0, st1, pos_nat, pos_v,
          gsem0, gsem1, ssem0, ssem1):
    c = lax.axis_index("c")
    s = lax.axis_index("s")
    wid = s * _NC + c
    j = wid // 4
    lane0 = (wid % 4) * _BPW

    rows = [rows0, rows1]
    stage = [st0, st1]
    gsem = [gsem0, gsem1]
    ssem = [ssem0, ssem1]

    pltpu.sync_copy(ids_hbm.at[:, j, :, pl.ds(lane0, _BPW)], idx_nat)
    pltpu.sync_copy(pos_hbm.at[:, pl.ds(0, 2), :, :], pos_nat)

    lane = jax.lax.iota(jnp.int32, 16)
    zero16 = lane * 0
    ge8 = jnp.where(lane >= 8, 1, 0)
    s_vec = lane - 8 * ge8

    def pos_body(l, carry):
        jt = l // 128
        ll = l % 128
        j_splat = zero16 + jt
        lane_splat = zero16 + ll
        for cc in range(_D // 16):
            i_vec = ge8 + 2 * cc
            v = plsc.load_gather(pos_nat, [i_vec, j_splat, s_vec, lane_splat])
            pos_v[l, pl.ds(cc * 16, 16)] = v
        return carry

    lax.fori_loop(0, _L, pos_body, 0)

    scatter_base = lane * _L

    def reorder_i(i, carry):
        for sub in range(8):
            l = i * 8 + sub
            for h in range(2):
                v = idx_nat[i, sub, pl.ds(h * 16, 16)]
                dst = scatter_base + (h * 16 * _L + l)
                plsc.store_scatter(idx_bm, [dst], v)
                plsc.store_scatter(idx_pair, [dst],
                                   lax.shift_right_logical(v, 1))
        return carry

    lax.fori_loop(0, 25, reorder_i, 0)

    def issue_gathers(g):
        hs = []
        for off, size in _SPLITS:
            hs.append(pltpu.async_copy(
                table_hbm.at[idx_pair.at[pl.ds(g * _L + off, size)]],
                rows[g % 2].at[pl.ds(off, size)],
                gsem[g % 2]))
        return hs

    gh = {0: issue_gathers(0)}
    sh = {}

    for g in range(_BPW):
        p = g % 2
        for h in gh[g]:
            h.wait()
        if g + 1 < _BPW:
            gh[g + 1] = issue_gathers(g + 1)
        if g >= 2:
            sh[g - 2].wait()

        rv = rows[p]
        sv = stage[p]
        gbase = g * _L

        def sel_body(r, carry):
            v = idx_bm[pl.ds(gbase + r, 16)][0]
            half = lax.rem(v, 2) * _D
            for jj in range(_D // 16):
                pv = pos_v[r, pl.ds(jj * 16, 16)]
                xv = rv[r, pl.ds(half + jj * 16, 16)]
                sv[r, pl.ds(jj * 16, 16)] = xv + pv
            return carry

        lax.fori_loop(0, _L, sel_body, 0)

        sh[g] = pltpu.async_copy(
            sv, out_hbm.at[wid * _BPW + g], ssem[p])

    sh[_BPW - 2].wait()
    sh[_BPW - 1].wait()


@jax.jit
def _embed(ids4, t128, pos4):
    mesh = plsc.VectorSubcoreMesh(core_axis_name="c", subcore_axis_name="s")
    f = pl.kernel(
        _body,
        out_type=jax.ShapeDtypeStruct((_B, _L, _D), jnp.float32),
        mesh=mesh,
        scratch_types=[
            pltpu.VMEM((_L // 8, 8, _BPW), jnp.int32),
            pltpu.VMEM((_BPW * _L + 16,), jnp.int32),
            pltpu.VMEM((_BPW * _L,), jnp.int32),
            pltpu.VMEM((_L, 2 * _D), jnp.float32),
            pltpu.VMEM((_L, 2 * _D), jnp.float32),
            pltpu.VMEM((_L, _D), jnp.float32),
            pltpu.VMEM((_L, _D), jnp.float32),
            pltpu.VMEM((8, 2, 8, 128), jnp.float32),
            pltpu.VMEM((_L, _D), jnp.float32),
            pltpu.SemaphoreType.DMA,
            pltpu.SemaphoreType.DMA,
            pltpu.SemaphoreType.DMA,
            pltpu.SemaphoreType.DMA,
        ],
        compiler_params=pltpu.CompilerParams(
            use_tc_tiling_on_sc=False, needs_layout_passes=False),
    )
    return f(ids4, t128, pos4)


def kernel(input_ids, token_table, pos_table):
    ids4 = (input_ids.astype(jnp.int32).T
            .reshape(_L // 8, 8, _B // 128, 128)
            .transpose(0, 2, 1, 3))
    ids4 = jnp.clip(ids4, 0, 999999)
    pos4 = (pos_table.T.reshape(8, 8, 16, 128).transpose(0, 2, 1, 3))
    pos4 = jnp.minimum(pos4, jnp.float32(3.4028235e38))
    t128 = token_table.reshape(500000, 128)
    return _embed(ids4, t128, pos4)


# speedup vs baseline: 1.2672x; 1.2672x over previous
import jax
import jax.numpy as jnp
from jax import lax
from jax.experimental import pallas as pl
from jax.experimental.pallas import tpu as pltpu
from jax.experimental.pallas import tpu_sc as plsc

_B = 1024
_L = 200
_D = 64
_N = _B * _L

_NC = 2
_NS = 16
_NW = _NC * _NS

_GB = 4
_GROUP_ROWS = _GB * _L
_SPLITS = ((0, 104), (104, 96))
_G = _B // (_NW * _GB)


def _body(ids_hbm, table_hbm, pos_hbm, out_hbm,
          idx0, idx1, rows0, rows1, pos_v,
          isem0, isem1, gsem0, gsem1, ssem0, ssem1):
    c = lax.axis_index("c")
    s = lax.axis_index("s")
    wid = s * _NC + c

    idx = [idx0, idx1]
    rows = [rows0, rows1]
    isem = [isem0, isem1]
    gsem = [gsem0, gsem1]
    ssem = [ssem0, ssem1]

    pltpu.sync_copy(pos_hbm.at[pl.ds(0, _L)], pos_v)

    base_batch = wid * (_G * _GB)

    def idx_copy(g):
        return pltpu.async_copy(
            ids_hbm.at[pl.ds(base_batch + g * _GB, _GB)],
            idx[g % 2], isem[g % 2])

    def issue_gathers(g):
        hs = []
        for b in range(_GB):
            for off, size in _SPLITS:
                hs.append(pltpu.async_copy(
                    table_hbm.at[idx[g % 2].at[b, pl.ds(off, size)]],
                    rows[g % 2].at[pl.ds(b * _L + off, size)],
                    gsem[g % 2]))
        return hs

    ih = {0: idx_copy(0), 1: idx_copy(1)}
    ih[0].wait()
    gh = {0: issue_gathers(0)}
    sh = {}

    for g in range(_G):
        p = g % 2
        for h in gh[g]:
            h.wait()
        if g + 1 < _G:
            if g >= 1:
                sh[g - 1].wait()
            ih[g + 1].wait()
            gh[g + 1] = issue_gathers(g + 1)
            if g + 2 < _G:
                ih[g + 2] = idx_copy(g + 2)

        rv = rows[p]

        def add_body(r, carry):
            for j in range(_D // 16):
                pv = pos_v[r, pl.ds(j * 16, 16)]
                for b in range(_GB):
                    plsc.addupdate(rv.at[b * _L + r, pl.ds(j * 16, 16)], pv)
            return carry

        lax.fori_loop(0, _L, add_body, 0)

        sh[g] = pltpu.async_copy(
            rv, out_hbm.at[pl.ds((wid * _G + g) * _GROUP_ROWS, _GROUP_ROWS)],
            ssem[p])

    sh[_G - 2].wait()
    sh[_G - 1].wait()


@jax.jit
def _embed(ids2d, token_table, pos_table):
    mesh = plsc.VectorSubcoreMesh(core_axis_name="c", subcore_axis_name="s")
    f = pl.kernel(
        _body,
        out_type=jax.ShapeDtypeStruct((_N, _D), jnp.float32),
        mesh=mesh,
        scratch_types=[
            pltpu.VMEM((_GB, _L), jnp.int32),
            pltpu.VMEM((_GB, _L), jnp.int32),
            pltpu.VMEM((_GROUP_ROWS, _D), jnp.float32),
            pltpu.VMEM((_GROUP_ROWS, _D), jnp.float32),
            pltpu.VMEM((_L, _D), jnp.float32),
            pltpu.SemaphoreType.DMA,
            pltpu.SemaphoreType.DMA,
            pltpu.SemaphoreType.DMA,
            pltpu.SemaphoreType.DMA,
            pltpu.SemaphoreType.DMA,
            pltpu.SemaphoreType.DMA,
        ],
        compiler_params=pltpu.CompilerParams(use_tc_tiling_on_sc=False),
    )
    return f(ids2d, token_table, pos_table)


def kernel(input_ids, token_table, pos_table):
    out = _embed(input_ids.astype(jnp.int32), token_table, pos_table)
    return out.reshape(_B, _L, _D)
